# baseline (speedup 1.0000x reference)
import functools
import math

import jax
import jax.numpy as jnp
import numpy as np
from jax import lax
from jax.experimental import pallas as pl
from jax.experimental.pallas import tpu as pltpu
from jax.experimental.pallas import tpu_sc as plsc

_NUM_WORKERS = 32
_LANES = 16
_BL = 128


def _positional_encoding(maxlen, dim):
    pos = jnp.arange(maxlen, dtype=jnp.float32)
    i = np.arange(dim)
    terms = jnp.asarray(1.0 / (10000.0 ** (2.0 * (i // 2) / float(dim))),
                        dtype=jnp.float32)
    pe_val = pos[:, None] * terms[None, :]
    pe = jnp.zeros((maxlen, dim), dtype=jnp.float32)
    pe = pe.at[:, 0::2].set(jnp.sin(pe_val[:, 0::2]))
    pe = pe.at[:, 1::2].set(jnp.cos(pe_val[:, 0::2]))
    return pe


@functools.partial(jax.jit, static_argnames=("b", "l"))
def _run(idx1d, W, pe, b, l):
    d = W.shape[1]
    dgs = d // 8
    n_bg = b // _BL
    n_slabs = l * n_bg // _NUM_WORKERS
    scale = 1.0 / math.sqrt(float(b))
    segs = d // _LANES
    mesh = plsc.VectorSubcoreMesh(core_axis_name="c", subcore_axis_name="s")

    @functools.partial(
        pl.kernel,
        mesh=mesh,
        out_type=jax.ShapeDtypeStruct((l, dgs, n_bg, 8, _BL), jnp.float32),
        scratch_types=[
            pltpu.VMEM((l, d), jnp.float32),
            pltpu.VMEM((_BL,), jnp.int32),
            pltpu.VMEM((_BL,), jnp.int32),
            pltpu.VMEM((_BL, d), jnp.float32),
            pltpu.VMEM((_BL, d), jnp.float32),
            pltpu.VMEM((d, _BL), jnp.float32),
            pltpu.VMEM((d, _BL), jnp.float32),
            pltpu.SemaphoreType.DMA,
            pltpu.SemaphoreType.DMA,
            pltpu.SemaphoreType.DMA,
            pltpu.SemaphoreType.DMA,
            pltpu.SemaphoreType.DMA,
            pltpu.SemaphoreType.DMA,
        ],
        compiler_params=pltpu.CompilerParams(use_tc_tiling_on_sc=False,
                                             needs_layout_passes=False),
    )
    def sc_kernel(w_hbm, idx_hbm, pe_hbm, out_hbm,
                  pe_v, i0, i1, g0, g1, o0, o1,
                  si0, si1, sg0, sg1, sw0, sw1):
        wid = lax.axis_index("s") * 2 + lax.axis_index("c")
        pltpu.sync_copy(pe_hbm, pe_v)

        iv, gb, ob = (i0, i1), (g0, g1), (o0, o1)
        si, sg, sw = (si0, si1), (sg0, sg1), (sw0, sw1)
        base = wid * n_slabs

        def slab_lg(s_):
            return s_ // n_bg, lax.rem(s_, n_bg)

        def issue_idx(k_, bi):
            pltpu.async_copy(idx_hbm.at[pl.ds((base + k_) * _BL, _BL)],
                             iv[bi], si[bi])

        def wait_idx(k_, bi):
            pltpu.make_async_copy(idx_hbm.at[pl.ds((base + k_) * _BL, _BL)],
                                  iv[bi], si[bi]).wait()

        def issue_gather(k_, bi):
            pltpu.async_copy(w_hbm.at[iv[bi]], gb[bi], sg[bi])

        def wait_gather(k_, bi):
            pltpu.make_async_copy(w_hbm.at[iv[bi]], gb[bi], sg[bi]).wait()

        def issue_wb(k_, bi):
            l_, bg_ = slab_lg(base + k_)
            for dg in range(dgs):
                pltpu.async_copy(ob[bi].at[pl.ds(dg * 8, 8)],
                                 out_hbm.at[l_, dg, bg_], sw[bi])

        def wait_wb(k_, bi):
            l_, bg_ = slab_lg(base + k_)
            for dg in range(dgs):
                pltpu.make_async_copy(ob[bi].at[pl.ds(dg * 8, 8)],
                                      out_hbm.at[l_, dg, bg_], sw[bi]).wait()

        def compute(k_, bi):
            l_, _ = slab_lg(base + k_)
            pes = [pe_v[l_, pl.ds(16 * j, _LANES)] for j in range(segs)]
            lane = lax.iota(jnp.int32, _LANES)
            dv = [lane + 16 * j for j in range(segs)]

            def row_body(bl, carry):
                blv = lax.broadcast_in_dim(bl, (_LANES,), ())
                for j in range(segs):
                    x = gb[bi][bl, pl.ds(16 * j, _LANES)]
                    y = x * scale + pes[j]
                    plsc.store_scatter(ob[bi], [dv[j], blv], y)
                return carry

            lax.fori_loop(0, _BL, row_body, 0, unroll=4)

        issue_idx(0, 0)
        issue_idx(1, 1)
        wait_idx(0, 0)
        issue_gather(0, 0)
        for k_ in (0, 1):
            wait_gather(k_, k_ % 2)
            issue_idx(k_ + 2, k_ % 2)
            compute(k_, k_ % 2)
            issue_wb(k_, k_ % 2)
            wait_idx(k_ + 1, (k_ + 1) % 2)
            issue_gather(k_ + 1, (k_ + 1) % 2)

        def group_body(g, carry):
            for bi in range(2):
                k_ = g * 2 + bi
                wait_wb(k_ - 2, bi)
                wait_gather(k_, bi)
                issue_idx(k_ + 2, bi)
                compute(k_, bi)
                issue_wb(k_, bi)
                wait_idx(k_ + 1, 1 - bi)
                issue_gather(k_ + 1, 1 - bi)
            return carry

        lax.fori_loop(1, n_slabs // 2 - 1, group_body, 0)

        k_ = n_slabs - 2
        wait_wb(k_ - 2, k_ % 2)
        wait_gather(k_, k_ % 2)
        compute(k_, k_ % 2)
        issue_wb(k_, k_ % 2)
        wait_idx(k_ + 1, (k_ + 1) % 2)
        issue_gather(k_ + 1, (k_ + 1) % 2)

        k_ = n_slabs - 1
        wait_wb(k_ - 2, k_ % 2)
        wait_gather(k_, k_ % 2)
        compute(k_, k_ % 2)
        issue_wb(k_, k_ % 2)

        wait_wb(n_slabs - 2, (n_slabs - 2) % 2)
        wait_wb(n_slabs - 1, (n_slabs - 1) % 2)

    return sc_kernel(W, idx1d, pe)


def kernel(inp, W):
    b, l = inp.shape
    d = W.shape[1]
    n_bg = b // _BL
    idx1d = inp.astype(jnp.int32).T.reshape(l * b)
    pe = _positional_encoding(l, d)
    out5d = _run(idx1d, W, pe, b, l)
    return out5d.transpose(2, 4, 0, 1, 3).reshape(b, l, d)


# speedup vs baseline: 1.0527x; 1.0527x over previous
import functools
import math

import jax
import jax.numpy as jnp
import numpy as np
from jax import lax
from jax.experimental import pallas as pl
from jax.experimental.pallas import tpu as pltpu
from jax.experimental.pallas import tpu_sc as plsc

_NUM_WORKERS = 32
_LANES = 16
_BL = 128


def _positional_encoding(maxlen, dim):
    pos = jnp.arange(maxlen, dtype=jnp.float32)
    i = np.arange(dim)
    terms = jnp.asarray(1.0 / (10000.0 ** (2.0 * (i // 2) / float(dim))),
                        dtype=jnp.float32)
    pe_val = pos[:, None] * terms[None, :]
    pe = jnp.zeros((maxlen, dim), dtype=jnp.float32)
    pe = pe.at[:, 0::2].set(jnp.sin(pe_val[:, 0::2]))
    pe = pe.at[:, 1::2].set(jnp.cos(pe_val[:, 0::2]))
    return pe


@functools.partial(jax.jit, static_argnames=("b", "l"))
def _run(idx1d, W, pe, b, l):
    d = W.shape[1]
    dgs = d // 8
    n_bg = b // _BL
    n_slabs = l * n_bg // _NUM_WORKERS
    scale = 1.0 / math.sqrt(float(b))
    segs = d // _LANES
    mesh = plsc.VectorSubcoreMesh(core_axis_name="c", subcore_axis_name="s")

    @functools.partial(
        pl.kernel,
        mesh=mesh,
        out_type=jax.ShapeDtypeStruct((l, dgs, n_bg, 8, _BL), jnp.float32),
        scratch_types=[
            pltpu.VMEM((l, d), jnp.float32),
            pltpu.VMEM((_BL,), jnp.int32),
            pltpu.VMEM((_BL,), jnp.int32),
            pltpu.VMEM((_BL, d), jnp.float32),
            pltpu.VMEM((_BL, d), jnp.float32),
            pltpu.VMEM((d, _BL), jnp.float32),
            pltpu.VMEM((d, _BL), jnp.float32),
            pltpu.SemaphoreType.DMA,
            pltpu.SemaphoreType.DMA,
            pltpu.SemaphoreType.DMA,
            pltpu.SemaphoreType.DMA,
            pltpu.SemaphoreType.DMA,
            pltpu.SemaphoreType.DMA,
        ],
        compiler_params=pltpu.CompilerParams(use_tc_tiling_on_sc=False,
                                             needs_layout_passes=False),
    )
    def sc_kernel(w_hbm, idx_hbm, pe_hbm, out_hbm,
                  pe_v, i0, i1, g0, g1, o0, o1,
                  si0, si1, sg0, sg1, sw0, sw1):
        wid = lax.axis_index("s") * 2 + lax.axis_index("c")
        pltpu.sync_copy(pe_hbm, pe_v)

        iv, gb, ob = (i0, i1), (g0, g1), (o0, o1)
        si, sg, sw = (si0, si1), (sg0, sg1), (sw0, sw1)
        base = wid * n_slabs

        def slab_lg(s_):
            return s_ // n_bg, lax.rem(s_, n_bg)

        def issue_idx(k_, bi):
            pltpu.async_copy(idx_hbm.at[pl.ds((base + k_) * _BL, _BL)],
                             iv[bi], si[bi])

        def wait_idx(k_, bi):
            pltpu.make_async_copy(idx_hbm.at[pl.ds((base + k_) * _BL, _BL)],
                                  iv[bi], si[bi]).wait()

        def issue_gather(k_, bi):
            pltpu.async_copy(w_hbm.at[iv[bi]], gb[bi], sg[bi])

        def wait_gather(k_, bi):
            pltpu.make_async_copy(w_hbm.at[iv[bi]], gb[bi], sg[bi]).wait()

        def issue_wb(k_, bi):
            l_, bg_ = slab_lg(base + k_)
            for dg in range(dgs):
                pltpu.async_copy(ob[bi].at[pl.ds(dg * 8, 8)],
                                 out_hbm.at[l_, dg, bg_], sw[bi])

        def wait_wb(k_, bi):
            l_, bg_ = slab_lg(base + k_)
            for dg in range(dgs):
                pltpu.make_async_copy(ob[bi].at[pl.ds(dg * 8, 8)],
                                      out_hbm.at[l_, dg, bg_], sw[bi]).wait()

        def compute(k_, bi):
            l_, _ = slab_lg(base + k_)
            pes = [pe_v[l_, pl.ds(16 * j, _LANES)] for j in range(segs)]
            lane = lax.iota(jnp.int32, _LANES)
            dv = [lane + 16 * j for j in range(segs)]

            def row_body(bl, carry):
                blv = lax.broadcast_in_dim(bl, (_LANES,), ())
                for j in range(segs):
                    x = gb[bi][bl, pl.ds(16 * j, _LANES)]
                    y = x * scale + pes[j]
                    plsc.store_scatter(ob[bi], [dv[j], blv], y)
                return carry

            lax.fori_loop(0, _BL, row_body, 0, unroll=4)

        issue_idx(0, 0)
        issue_idx(1, 1)
        wait_idx(0, 0)
        issue_gather(0, 0)
        for k_ in (0, 1):
            wait_gather(k_, k_ % 2)
            issue_idx(k_ + 2, k_ % 2)
            wait_idx(k_ + 1, (k_ + 1) % 2)
            issue_gather(k_ + 1, (k_ + 1) % 2)
            compute(k_, k_ % 2)
            issue_wb(k_, k_ % 2)

        def group_body(g, carry):
            for bi in range(2):
                k_ = g * 2 + bi
                wait_wb(k_ - 2, bi)
                wait_gather(k_, bi)
                issue_idx(k_ + 2, bi)
                wait_idx(k_ + 1, 1 - bi)
                issue_gather(k_ + 1, 1 - bi)
                compute(k_, bi)
                issue_wb(k_, bi)
            return carry

        lax.fori_loop(1, n_slabs // 2 - 1, group_body, 0)

        k_ = n_slabs - 2
        wait_wb(k_ - 2, k_ % 2)
        wait_gather(k_, k_ % 2)
        wait_idx(k_ + 1, (k_ + 1) % 2)
        issue_gather(k_ + 1, (k_ + 1) % 2)
        compute(k_, k_ % 2)
        issue_wb(k_, k_ % 2)

        k_ = n_slabs - 1
        wait_wb(k_ - 2, k_ % 2)
        wait_gather(k_, k_ % 2)
        compute(k_, k_ % 2)
        issue_wb(k_, k_ % 2)

        wait_wb(n_slabs - 2, (n_slabs - 2) % 2)
        wait_wb(n_slabs - 1, (n_slabs - 1) % 2)

    return sc_kernel(W, idx1d, pe)


def kernel(inp, W):
    b, l = inp.shape
    d = W.shape[1]
    n_bg = b // _BL
    idx1d = jnp.clip(inp.astype(jnp.int32).T.reshape(l * b), 0, W.shape[0] - 1)
    pe = _positional_encoding(l, d)
    out5d = _run(idx1d, W, pe, b, l)
    return out5d.transpose(2, 4, 0, 1, 3).reshape(b, l, d)


# speedup vs baseline: 1.2174x; 1.1564x over previous
import functools
import math

import jax
import jax.numpy as jnp
import numpy as np
from jax import lax
from jax.experimental import pallas as pl
from jax.experimental.pallas import tpu as pltpu
from jax.experimental.pallas import tpu_sc as plsc

_NUM_WORKERS = 32
_LANES = 16
_BL = 128


def _positional_encoding(maxlen, dim):
    pos = jnp.arange(maxlen, dtype=jnp.float32)
    i = np.arange(dim)
    terms = jnp.asarray(1.0 / (10000.0 ** (2.0 * (i // 2) / float(dim))),
                        dtype=jnp.float32)
    pe_val = pos[:, None] * terms[None, :]
    pe = jnp.zeros((maxlen, dim), dtype=jnp.float32)
    pe = pe.at[:, 0::2].set(jnp.sin(pe_val[:, 0::2]))
    pe = pe.at[:, 1::2].set(jnp.cos(pe_val[:, 0::2]))
    return pe


@functools.partial(jax.jit, static_argnames=("b", "l"))
def _run(idx1d, W, pe, b, l):
    d = W.shape[1]
    dgs = d // 8
    n_bg = b // _BL
    n_slabs = l * n_bg // _NUM_WORKERS
    scale = 1.0 / math.sqrt(float(b))
    segs = d // _LANES
    mesh = plsc.VectorSubcoreMesh(core_axis_name="c", subcore_axis_name="s")

    @functools.partial(
        pl.kernel,
        mesh=mesh,
        out_type=jax.ShapeDtypeStruct((l, dgs, n_bg, 8, _BL), jnp.float32),
        scratch_types=[
            pltpu.VMEM((l, d), jnp.float32),
            pltpu.VMEM((_BL,), jnp.int32),
            pltpu.VMEM((_BL,), jnp.int32),
            pltpu.VMEM((_BL, d), jnp.float32),
            pltpu.VMEM((_BL, d), jnp.float32),
            pltpu.VMEM((d, _BL), jnp.float32),
            pltpu.VMEM((d, _BL), jnp.float32),
            pltpu.SemaphoreType.DMA,
            pltpu.SemaphoreType.DMA,
            pltpu.SemaphoreType.DMA,
            pltpu.SemaphoreType.DMA,
            pltpu.SemaphoreType.DMA,
            pltpu.SemaphoreType.DMA,
        ],
        compiler_params=pltpu.CompilerParams(use_tc_tiling_on_sc=False,
                                             needs_layout_passes=False),
    )
    def sc_kernel(w_hbm, idx_hbm, pe_hbm, out_hbm,
                  pe_v, i0, i1, g0, g1, o0, o1,
                  si0, si1, sg0, sg1, sw0, sw1):
        wid = lax.axis_index("s") * 2 + lax.axis_index("c")
        pltpu.sync_copy(pe_hbm, pe_v)

        iv, gb, ob = (i0, i1), (g0, g1), (o0, o1)
        si, sg, sw = (si0, si1), (sg0, sg1), (sw0, sw1)
        base = wid * n_slabs

        def slab_lg(s_):
            return s_ // n_bg, lax.rem(s_, n_bg)

        def issue_idx(k_, bi):
            pltpu.async_copy(idx_hbm.at[pl.ds((base + k_) * _BL, _BL)],
                             iv[bi], si[bi])

        def wait_idx(k_, bi):
            pltpu.make_async_copy(idx_hbm.at[pl.ds((base + k_) * _BL, _BL)],
                                  iv[bi], si[bi]).wait()

        def issue_gather(k_, bi):
            pltpu.async_copy(w_hbm.at[iv[bi]], gb[bi], sg[bi])

        def wait_gather(k_, bi):
            pltpu.make_async_copy(w_hbm.at[iv[bi]], gb[bi], sg[bi]).wait()

        def issue_wb(k_, bi):
            l_, bg_ = slab_lg(base + k_)
            for dg in range(dgs):
                pltpu.async_copy(ob[bi].at[pl.ds(dg * 8, 8)],
                                 out_hbm.at[l_, dg, bg_], sw[bi])

        def wait_wb(k_, bi):
            l_, bg_ = slab_lg(base + k_)
            for dg in range(dgs):
                pltpu.make_async_copy(ob[bi].at[pl.ds(dg * 8, 8)],
                                      out_hbm.at[l_, dg, bg_], sw[bi]).wait()

        def compute(k_, bi):
            l_, _ = slab_lg(base + k_)
            pes = [pe_v[l_, pl.ds(16 * j, _LANES)] for j in range(segs)]
            lane = lax.iota(jnp.int32, _LANES)
            dv = [lane + 16 * j for j in range(segs)]

            @plsc.parallel_loop(0, _BL, unroll=8)
            def row_body(bl):
                blv = lax.broadcast_in_dim(bl, (_LANES,), ())
                for j in range(segs):
                    x = gb[bi][bl, pl.ds(16 * j, _LANES)]
                    y = x * scale + pes[j]
                    plsc.store_scatter(ob[bi], [dv[j], blv], y)

        issue_idx(0, 0)
        issue_idx(1, 1)
        wait_idx(0, 0)
        issue_gather(0, 0)
        for k_ in (0, 1):
            wait_gather(k_, k_ % 2)
            issue_idx(k_ + 2, k_ % 2)
            wait_idx(k_ + 1, (k_ + 1) % 2)
            issue_gather(k_ + 1, (k_ + 1) % 2)
            compute(k_, k_ % 2)
            issue_wb(k_, k_ % 2)

        def group_body(g, carry):
            for bi in range(2):
                k_ = g * 2 + bi
                wait_wb(k_ - 2, bi)
                wait_gather(k_, bi)
                issue_idx(k_ + 2, bi)
                wait_idx(k_ + 1, 1 - bi)
                issue_gather(k_ + 1, 1 - bi)
                compute(k_, bi)
                issue_wb(k_, bi)
            return carry

        lax.fori_loop(1, n_slabs // 2 - 1, group_body, 0)

        k_ = n_slabs - 2
        wait_wb(k_ - 2, k_ % 2)
        wait_gather(k_, k_ % 2)
        wait_idx(k_ + 1, (k_ + 1) % 2)
        issue_gather(k_ + 1, (k_ + 1) % 2)
        compute(k_, k_ % 2)
        issue_wb(k_, k_ % 2)

        k_ = n_slabs - 1
        wait_wb(k_ - 2, k_ % 2)
        wait_gather(k_, k_ % 2)
        compute(k_, k_ % 2)
        issue_wb(k_, k_ % 2)

        wait_wb(n_slabs - 2, (n_slabs - 2) % 2)
        wait_wb(n_slabs - 1, (n_slabs - 1) % 2)

    return sc_kernel(W, idx1d, pe)


def kernel(inp, W):
    b, l = inp.shape
    d = W.shape[1]
    n_bg = b // _BL
    idx1d = jnp.clip(inp.astype(jnp.int32).T.reshape(l * b), 0, W.shape[0] - 1)
    pe = _positional_encoding(l, d)
    out5d = _run(idx1d, W, pe, b, l)
    return out5d.transpose(2, 4, 0, 1, 3).reshape(b, l, d)
